# baseline (speedup 1.0000x reference)
import functools

import jax
import jax.numpy as jnp
from jax.experimental import pallas as pl
from jax.experimental.pallas import tpu as pltpu


def _expert_mlp_kernel(xb_ref, wi_ref, wo_ref, m0_ref, m1_ref, r0_ref, r1_ref,
                       o_ref, xc_ref, *, bt, nt):
    e = pl.program_id(0)
    f = pl.program_id(1)
    first = (e == 0) & (f == 0)

    @pl.when(f == 0)
    def _():
        call = (m0_ref[...] * r0_ref[...] + m1_ref[...] * r1_ref[...])
        onehot = jax.lax.broadcasted_iota(jnp.int32, call.shape, 1) == e
        c = jnp.sum(jnp.where(onehot, call, 0.0), axis=1, keepdims=True)
        xc_ref[...] = (xb_ref[...].astype(jnp.float32) * c).astype(jnp.bfloat16)

    wib = wi_ref[0].astype(jnp.bfloat16)
    wob = wo_ref[0].astype(jnp.bfloat16)

    for t in range(nt):
        rows = pl.ds(t * bt, bt)
        x = xc_ref[rows, :]
        h = jax.lax.dot_general(x, wib, (((1,), (1,)), ((), ())),
                                preferred_element_type=jnp.float32)
        h = jnp.maximum(h, 0.0).astype(jnp.bfloat16)
        o = jax.lax.dot_general(h, wob, (((1,), (1,)), ((), ())),
                                preferred_element_type=jnp.float32)

        @pl.when(first)
        def _():
            o_ref[rows, :] = o

        @pl.when(jnp.logical_not(first))
        def _():
            o_ref[rows, :] += o


def kernel(hidden_states, selected_experts, routing_weights, wi, wo):
    T, D = hidden_states.shape
    E, F, _ = wi.shape

    xb = hidden_states.astype(jnp.bfloat16)
    maskf = selected_experts.astype(jnp.float32)
    m0 = maskf[:, 0, :]
    m1 = maskf[:, 1, :]
    r0 = routing_weights[:, 0:1]
    r1 = routing_weights[:, 1:2]

    BT = 256
    BF = 1536
    NT = T // BT
    NF = F // BF

    body = functools.partial(_expert_mlp_kernel, bt=BT, nt=NT)

    out = pl.pallas_call(
        body,
        grid=(E, NF),
        in_specs=[
            pl.BlockSpec((T, D), lambda e, f: (0, 0)),
            pl.BlockSpec((1, BF, D), lambda e, f: (e, f, 0)),
            pl.BlockSpec((1, D, BF), lambda e, f: (e, 0, f)),
            pl.BlockSpec((T, E), lambda e, f: (0, 0)),
            pl.BlockSpec((T, E), lambda e, f: (0, 0)),
            pl.BlockSpec((T, 1), lambda e, f: (0, 0)),
            pl.BlockSpec((T, 1), lambda e, f: (0, 0)),
        ],
        out_specs=pl.BlockSpec((T, D), lambda e, f: (0, 0)),
        out_shape=jax.ShapeDtypeStruct((T, D), jnp.float32),
        scratch_shapes=[pltpu.VMEM((T, D), jnp.bfloat16)],
    )(xb, wi, wo, m0, m1, r0, r1)
    return out


# speedup vs baseline: 1.2819x; 1.2819x over previous
import functools

import jax
import jax.numpy as jnp
from jax.experimental import pallas as pl
from jax.experimental.pallas import tpu as pltpu


def _expert_mlp_kernel(xb_ref, wi_ref, wo_ref, m0_ref, m1_ref, r0_ref, r1_ref,
                       o_ref, *, bt, nt):
    e = pl.program_id(0)
    f = pl.program_id(1)
    first = (e == 0) & (f == 0)

    wib = wi_ref[0].astype(jnp.bfloat16)
    wob = wo_ref[0].astype(jnp.bfloat16)

    for t in range(nt):
        rows = pl.ds(t * bt, bt)
        x = xb_ref[rows, :]
        h = jax.lax.dot_general(x, wib, (((1,), (1,)), ((), ())),
                                preferred_element_type=jnp.float32)
        h = jnp.maximum(h, 0.0).astype(jnp.bfloat16)
        o = jax.lax.dot_general(h, wob, (((1,), (1,)), ((), ())),
                                preferred_element_type=jnp.float32)

        call = (m0_ref[rows, :] * r0_ref[rows, :]
                + m1_ref[rows, :] * r1_ref[rows, :])
        onehot = jax.lax.broadcasted_iota(jnp.int32, call.shape, 1) == e
        c = jnp.sum(jnp.where(onehot, call, 0.0), axis=1, keepdims=True)
        contrib = o * c

        @pl.when(first)
        def _():
            o_ref[rows, :] = contrib

        @pl.when(jnp.logical_not(first))
        def _():
            o_ref[rows, :] += contrib


def kernel(hidden_states, selected_experts, routing_weights, wi, wo):
    T, D = hidden_states.shape
    E, F, _ = wi.shape

    xb = hidden_states.astype(jnp.bfloat16)
    maskf = selected_experts.astype(jnp.float32)
    m0 = maskf[:, 0, :]
    m1 = maskf[:, 1, :]
    r0 = routing_weights[:, 0:1]
    r1 = routing_weights[:, 1:2]

    BT = 1024
    BF = 1536
    NT = T // BT
    NF = F // BF

    body = functools.partial(_expert_mlp_kernel, bt=BT, nt=NT)

    out = pl.pallas_call(
        body,
        grid=(E, NF),
        in_specs=[
            pl.BlockSpec((T, D), lambda e, f: (0, 0)),
            pl.BlockSpec((1, BF, D), lambda e, f: (e, f, 0)),
            pl.BlockSpec((1, D, BF), lambda e, f: (e, 0, f)),
            pl.BlockSpec((T, E), lambda e, f: (0, 0)),
            pl.BlockSpec((T, E), lambda e, f: (0, 0)),
            pl.BlockSpec((T, 1), lambda e, f: (0, 0)),
            pl.BlockSpec((T, 1), lambda e, f: (0, 0)),
        ],
        out_specs=pl.BlockSpec((T, D), lambda e, f: (0, 0)),
        out_shape=jax.ShapeDtypeStruct((T, D), jnp.float32),
    )(xb, wi, wo, m0, m1, r0, r1)
    return out


# speedup vs baseline: 1.4087x; 1.0988x over previous
import functools

import jax
import jax.numpy as jnp
from jax.experimental import pallas as pl
from jax.experimental.pallas import tpu as pltpu


def _expert_mlp_kernel(xb_ref, wi_ref, wo_ref, m0_ref, m1_ref, r0_ref, r1_ref,
                       o_ref, *, bt, nt):
    e = pl.program_id(0)
    f = pl.program_id(1)

    @pl.when((e == 0) & (f == 0))
    def _():
        o_ref[...] = jnp.zeros_like(o_ref)

    wib = wi_ref[0].astype(jnp.bfloat16)
    wob = wo_ref[0].astype(jnp.bfloat16)

    for t in range(nt):
        rows = pl.ds(t * bt, bt)
        x = xb_ref[rows, :]
        h = jax.lax.dot_general(x, wib, (((1,), (1,)), ((), ())),
                                preferred_element_type=jnp.float32)
        h = jnp.maximum(h, 0.0).astype(jnp.bfloat16)
        o = jax.lax.dot_general(h, wob, (((1,), (1,)), ((), ())),
                                preferred_element_type=jnp.float32)

        call = (m0_ref[rows, :] * r0_ref[rows, :]
                + m1_ref[rows, :] * r1_ref[rows, :])
        onehot = jax.lax.broadcasted_iota(jnp.int32, call.shape, 1) == e
        c = jnp.sum(jnp.where(onehot, call, 0.0), axis=1, keepdims=True)
        o_ref[rows, :] += o * c


def kernel(hidden_states, selected_experts, routing_weights, wi, wo):
    T, D = hidden_states.shape
    E, F, _ = wi.shape

    xb = hidden_states.astype(jnp.bfloat16)
    maskf = selected_experts.astype(jnp.float32)
    m0 = maskf[:, 0, :]
    m1 = maskf[:, 1, :]
    r0 = routing_weights[:, 0:1]
    r1 = routing_weights[:, 1:2]

    BT = 1024
    BF = 1536
    NT = T // BT
    NF = F // BF

    body = functools.partial(_expert_mlp_kernel, bt=BT, nt=NT)

    out = pl.pallas_call(
        body,
        grid=(E, NF),
        in_specs=[
            pl.BlockSpec((T, D), lambda e, f: (0, 0)),
            pl.BlockSpec((1, BF, D), lambda e, f: (e, f, 0)),
            pl.BlockSpec((1, D, BF), lambda e, f: (e, 0, f)),
            pl.BlockSpec((T, E), lambda e, f: (0, 0)),
            pl.BlockSpec((T, E), lambda e, f: (0, 0)),
            pl.BlockSpec((T, 1), lambda e, f: (0, 0)),
            pl.BlockSpec((T, 1), lambda e, f: (0, 0)),
        ],
        out_specs=pl.BlockSpec((T, D), lambda e, f: (0, 0)),
        out_shape=jax.ShapeDtypeStruct((T, D), jnp.float32),
    )(xb, wi, wo, m0, m1, r0, r1)
    return out


# speedup vs baseline: 1.4176x; 1.0063x over previous
import functools

import jax
import jax.numpy as jnp
from jax.experimental import pallas as pl
from jax.experimental.pallas import tpu as pltpu


def _expert_mlp_kernel(xb_ref, wi_ref, wo_ref, m0_ref, m1_ref, r0_ref, r1_ref,
                       o_ref, *, bt, nt):
    e = pl.program_id(0)
    f = pl.program_id(1)

    @pl.when((e == 0) & (f == 0))
    def _():
        o_ref[...] = jnp.zeros_like(o_ref)

    wib = wi_ref[0].astype(jnp.bfloat16)
    wob = wo_ref[0].astype(jnp.bfloat16)

    def mm1(t):
        rows = pl.ds(t * bt, bt)
        x = xb_ref[rows, :]
        h = jax.lax.dot_general(x, wib, (((1,), (1,)), ((), ())),
                                preferred_element_type=jnp.float32)
        return jnp.maximum(h, 0.0).astype(jnp.bfloat16)

    def mm2_accum(t, h):
        rows = pl.ds(t * bt, bt)
        o = jax.lax.dot_general(h, wob, (((1,), (1,)), ((), ())),
                                preferred_element_type=jnp.float32)
        call = (m0_ref[rows, :] * r0_ref[rows, :]
                + m1_ref[rows, :] * r1_ref[rows, :])
        onehot = jax.lax.broadcasted_iota(jnp.int32, call.shape, 1) == e
        c = jnp.sum(jnp.where(onehot, call, 0.0), axis=1, keepdims=True)
        o_ref[rows, :] += o * c

    h_prev = mm1(0)
    for t in range(1, nt):
        h_cur = mm1(t)
        mm2_accum(t - 1, h_prev)
        h_prev = h_cur
    mm2_accum(nt - 1, h_prev)


def kernel(hidden_states, selected_experts, routing_weights, wi, wo):
    T, D = hidden_states.shape
    E, F, _ = wi.shape

    xb = hidden_states.astype(jnp.bfloat16)
    maskf = selected_experts.astype(jnp.float32)
    m0 = maskf[:, 0, :]
    m1 = maskf[:, 1, :]
    r0 = routing_weights[:, 0:1]
    r1 = routing_weights[:, 1:2]

    BT = 1024
    BF = 1536
    NT = T // BT
    NF = F // BF

    body = functools.partial(_expert_mlp_kernel, bt=BT, nt=NT)

    out = pl.pallas_call(
        body,
        grid=(E, NF),
        in_specs=[
            pl.BlockSpec((T, D), lambda e, f: (0, 0)),
            pl.BlockSpec((1, BF, D), lambda e, f: (e, f, 0)),
            pl.BlockSpec((1, D, BF), lambda e, f: (e, 0, f)),
            pl.BlockSpec((T, E), lambda e, f: (0, 0)),
            pl.BlockSpec((T, E), lambda e, f: (0, 0)),
            pl.BlockSpec((T, 1), lambda e, f: (0, 0)),
            pl.BlockSpec((T, 1), lambda e, f: (0, 0)),
        ],
        out_specs=pl.BlockSpec((T, D), lambda e, f: (0, 0)),
        out_shape=jax.ShapeDtypeStruct((T, D), jnp.float32),
    )(xb, wi, wo, m0, m1, r0, r1)
    return out
